# baseline (speedup 1.0000x reference)
import jax
import jax.numpy as jnp
from jax.experimental import pallas as pl


def _head(h_ref, W1_ref, b1_ref, W2_ref, b2_ref, W3_ref, b3_ref, o_ref):
    hg = jnp.mean(h_ref[...], axis=0, keepdims=True)
    hg = jax.nn.relu(jnp.dot(hg, W1_ref[...], preferred_element_type=jnp.float32) + b1_ref[...])
    hg = jax.nn.relu(jnp.dot(hg, W2_ref[...], preferred_element_type=jnp.float32) + b2_ref[...])
    o_ref[...] = jnp.dot(hg, W3_ref[...], preferred_element_type=jnp.float32) + b3_ref[...]


def kernel(h, e, edge_index, Wn, Wn_b, We, We_b, A, A_b, Bm, B_b, Cm, C_b, Dm, D_b, Em, E_b, gh, bh, ge, be, W1, b1, W2, b2, W3, b3):
    src = edge_index[0]
    dst = edge_index[1]
    n_nodes = h.shape[0]
    h = h @ Wn + Wn_b
    e = e @ We + We_b
    degs = jnp.clip(jax.ops.segment_sum(jnp.ones((src.shape[0],), dtype=jnp.float32), dst, num_segments=n_nodes), 1.0, None)
    norm = (degs ** -0.5)[:, None]
    L = A.shape[0]
    for i in range(L):
        h_in = h
        e_in = e
        Ah = h @ A[i] + A_b[i]
        Bh = h @ Bm[i] + B_b[i]
        Dh = h @ Dm[i] + D_b[i]
        Eh = h @ Em[i] + E_b[i]
        e_new = e @ Cm[i] + C_b[i] + Dh[src] + Eh[dst]
        sigma = jax.nn.sigmoid(e_new)
        msg = sigma * (Bh * norm)[src]
        h_new = Ah + norm * jax.ops.segment_sum(msg, dst, num_segments=n_nodes)
        mu_h = h_new.mean(axis=0)
        var_h = h_new.var(axis=0)
        h_bn = (h_new - mu_h) / jnp.sqrt(var_h + 1e-5) * gh[i] + bh[i]
        mu_e = e_new.mean(axis=0)
        var_e = e_new.var(axis=0)
        e_bn = (e_new - mu_e) / jnp.sqrt(var_e + 1e-5) * ge[i] + be[i]
        h = jax.nn.relu(h_bn) + h_in
        e = jax.nn.relu(e_bn) + e_in
    return pl.pallas_call(
        _head,
        out_shape=jax.ShapeDtypeStruct((1, W3.shape[1]), jnp.float32),
    )(h, W1, b1, W2, b2, W3, b3)
